# baseline (speedup 1.0000x reference)
import functools

import jax
import jax.numpy as jnp
from jax import lax
from jax.experimental import pallas as pl
from jax.experimental.pallas import tpu as pltpu
from jax.experimental.pallas import tpu_sc as plsc

BATCH = 4096
HIST = 200
DIM = 32
B = BATCH * HIST

_info = plsc.get_sparse_core_info()
NC, NS = _info.num_cores, _info.num_subcores
NW = NC * NS
B_PER_W = B // NW

UNIT = 256
TB_PER_UNIT = UNIT // 128
NUNITS = B_PER_W // UNIT
NTB = BATCH // 128
NTD = DIM // 8
UNITS_PER_H = NTB // TB_PER_UNIT
OUT_ROWS = B * DIM // 128


def _make_emb():
    mesh = plsc.VectorSubcoreMesh(core_axis_name="c", subcore_axis_name="s")

    @functools.partial(
        pl.kernel,
        mesh=mesh,
        out_type=jax.ShapeDtypeStruct((OUT_ROWS, 128), jnp.float32),
        scratch_types=[
            pltpu.VMEM((B_PER_W,), jnp.int32),
            pltpu.VMEM((2, UNIT, DIM), jnp.float32),
            pltpu.VMEM((2, NTD, TB_PER_UNIT * 8, 128), jnp.float32),
            pltpu.SemaphoreType.DMA,
            pltpu.SemaphoreType.DMA,
            pltpu.SemaphoreType.DMA,
            pltpu.SemaphoreType.DMA,
            pltpu.SemaphoreType.DMA,
        ],
        compiler_params=pltpu.CompilerParams(
            use_tc_tiling_on_sc=False, needs_layout_passes=False
        ),
    )
    def emb(idx_hbm, table_hbm, out_hbm, idx_all, grows, trows, isem, g0, g1, o0, o1):
        gsems = (g0, g1)
        osems = (o0, o1)
        wid = lax.axis_index("s") * NC + lax.axis_index("c")
        wbase = wid * B_PER_W
        ubase = wid * NUNITS

        pltpu.async_copy(idx_hbm.at[pl.ds(wbase, B_PER_W)], idx_all, isem).wait()

        def start_gather(j, buf):
            pltpu.async_copy(
                table_hbm.at[idx_all.at[pl.ds(j * UNIT, UNIT)]],
                grows.at[buf],
                gsems[buf],
            )

        def wait_gather(buf):
            pltpu.make_async_copy(
                table_hbm.at[idx_all.at[pl.ds(0, UNIT)]],
                grows.at[buf],
                gsems[buf],
            ).wait()

        def transpose(buf):
            g = grows.at[buf]
            iota = lax.iota(jnp.int32, 16)
            for td in range(NTD):
                for r in range(8):
                    cols = jnp.full((16,), td * 8 + r, jnp.int32)
                    for tbl in range(TB_PER_UNIT):
                        for c0 in range(0, 128, 16):
                            rows = iota + (tbl * 128 + c0)
                            v = plsc.load_gather(g, [rows, cols])
                            trows[buf, td, tbl * 8 + r, pl.ds(c0, 16)] = v

        def start_stores(j, buf):
            u = ubase + j
            h = u // UNITS_PER_H
            tbq = u % UNITS_PER_H
            for td in range(NTD):
                row0 = h * 1024 + td * 256 + tbq * (TB_PER_UNIT * 8)
                pltpu.async_copy(
                    trows.at[buf, td],
                    out_hbm.at[pl.ds(row0, TB_PER_UNIT * 8)],
                    osems[buf],
                )

        def wait_stores(buf):
            for _ in range(NTD):
                pltpu.make_async_copy(
                    trows.at[buf, 0],
                    out_hbm.at[pl.ds(0, TB_PER_UNIT * 8)],
                    osems[buf],
                ).wait()

        start_gather(0, 0)
        start_gather(1, 1)

        def body(jj, carry):
            for buf in range(2):
                j = 2 * jj + buf
                wait_gather(buf)

                @pl.when(jj >= 1)
                def _():
                    wait_stores(buf)

                transpose(buf)

                @pl.when(jj < NUNITS // 2 - 1)
                def _():
                    start_gather(j + 2, buf)

                start_stores(j, buf)
            return carry

        lax.fori_loop(0, NUNITS // 2, body, 0)

        wait_stores(0)
        wait_stores(1)

    return emb


_emb = _make_emb()


def kernel(tokens, table):
    idx = tokens.T.reshape(B).astype(jnp.int32)
    out = _emb(idx, table)
    out6 = out.reshape(HIST, NTD, NTB, 8, 128)
    return out6.transpose(2, 4, 0, 1, 3).reshape(BATCH, HIST, DIM)


# speedup vs baseline: 1.6864x; 1.6864x over previous
import functools

import jax
import jax.numpy as jnp
from jax import lax
from jax.experimental import pallas as pl
from jax.experimental.pallas import tpu as pltpu
from jax.experimental.pallas import tpu_sc as plsc

BATCH = 4096
HIST = 200
DIM = 32
B = BATCH * HIST

_info = plsc.get_sparse_core_info()
NC, NS = _info.num_cores, _info.num_subcores
NW = NC * NS
B_PER_W = B // NW

UNIT = 512
TB_PER_UNIT = UNIT // 128
NUNITS = B_PER_W // UNIT
NTB = BATCH // 128
NTD = DIM // 8
UNITS_PER_H = NTB // TB_PER_UNIT
OUT_ROWS = B * DIM // 128


def _make_emb():
    mesh = plsc.VectorSubcoreMesh(core_axis_name="c", subcore_axis_name="s")

    @functools.partial(
        pl.kernel,
        mesh=mesh,
        out_type=jax.ShapeDtypeStruct((OUT_ROWS, 128), jnp.float32),
        scratch_types=[
            pltpu.VMEM((B_PER_W,), jnp.int32),
            pltpu.VMEM((2, UNIT, DIM), jnp.float32),
            pltpu.VMEM((2, NTD, 40, 129), jnp.float32),
            pltpu.SemaphoreType.DMA,
            pltpu.SemaphoreType.DMA,
            pltpu.SemaphoreType.DMA,
            pltpu.SemaphoreType.DMA,
            pltpu.SemaphoreType.DMA,
        ],
        compiler_params=pltpu.CompilerParams(
            use_tc_tiling_on_sc=False, needs_layout_passes=False
        ),
    )
    def emb(idx_hbm, table_hbm, out_hbm, idx_all, grows, trows, isem, g0, g1, o0, o1):
        gsems = (g0, g1)
        osems = (o0, o1)
        wid = lax.axis_index("s") * NC + lax.axis_index("c")
        wbase = wid * B_PER_W
        ubase = wid * NUNITS

        pltpu.async_copy(idx_hbm.at[pl.ds(wbase, B_PER_W)], idx_all, isem).wait()

        def start_gather(j, buf):
            pltpu.async_copy(
                table_hbm.at[idx_all.at[pl.ds(j * UNIT, UNIT)]],
                grows.at[buf],
                gsems[buf],
            )

        def wait_gather(buf):
            pltpu.make_async_copy(
                table_hbm.at[idx_all.at[pl.ds(0, UNIT)]],
                grows.at[buf],
                gsems[buf],
            ).wait()

        def transpose(buf):
            t_ref = trows.at[buf]
            iota = lax.iota(jnp.int32, 16)
            td_lo = jnp.bitwise_and(jnp.right_shift(iota, 3), 1)
            td_hi = td_lo + 2
            dmod = jnp.bitwise_and(iota, 7)
            zv = jnp.bitwise_and(iota, 0)

            def col(c, carry):
                cv = zv + c
                for tbl in range(TB_PER_UNIT):
                    rowv = dmod + (tbl * 8)
                    t = tbl * 128 + c
                    v0 = grows[buf, t, pl.ds(0, 16)]
                    plsc.store_scatter(t_ref, [td_lo, rowv, cv], v0)
                    v1 = grows[buf, t, pl.ds(16, 16)]
                    plsc.store_scatter(t_ref, [td_hi, rowv, cv], v1)
                return carry

            lax.fori_loop(0, 128, col, 0)

        def start_stores(j, buf):
            u = ubase + j
            h = u // UNITS_PER_H
            tbq = u % UNITS_PER_H
            for td in range(NTD):
                row0 = h * 1024 + td * 256 + tbq * (TB_PER_UNIT * 8)
                pltpu.async_copy(
                    trows.at[buf, td, pl.ds(0, TB_PER_UNIT * 8), pl.ds(0, 128)],
                    out_hbm.at[pl.ds(row0, TB_PER_UNIT * 8)],
                    osems[buf],
                )

        def wait_stores(buf):
            for _ in range(NTD):
                pltpu.make_async_copy(
                    trows.at[buf, 0, pl.ds(0, TB_PER_UNIT * 8), pl.ds(0, 128)],
                    out_hbm.at[pl.ds(0, TB_PER_UNIT * 8)],
                    osems[buf],
                ).wait()

        start_gather(0, 0)
        start_gather(1, 1)

        def body(jj, carry):
            for buf in range(2):
                j = 2 * jj + buf
                wait_gather(buf)

                @pl.when(jj >= 1)
                def _():
                    wait_stores(buf)

                transpose(buf)

                @pl.when(jj < NUNITS // 2 - 1)
                def _():
                    start_gather(j + 2, buf)

                start_stores(j, buf)
            return carry

        lax.fori_loop(0, NUNITS // 2, body, 0)

        wait_stores(0)
        wait_stores(1)

    return emb


_emb = _make_emb()

def kernel(tokens, table):
    idx = tokens.T.reshape(B).astype(jnp.int32)
    out = _emb(idx, table)
    out6 = out.reshape(HIST, NTD, NTB, 8, 128)
    return out6.transpose(2, 4, 0, 1, 3).reshape(BATCH, HIST, DIM)


# speedup vs baseline: 1.6959x; 1.0056x over previous
import functools

import jax
import jax.numpy as jnp
from jax import lax
from jax.experimental import pallas as pl
from jax.experimental.pallas import tpu as pltpu
from jax.experimental.pallas import tpu_sc as plsc

BATCH = 4096
HIST = 200
DIM = 32
B = BATCH * HIST

_info = plsc.get_sparse_core_info()
NC, NS = _info.num_cores, _info.num_subcores
NW = NC * NS
B_PER_W = B // NW

UNIT = 128
TB_PER_UNIT = UNIT // 128
NUNITS = B_PER_W // UNIT
NTB = BATCH // 128
NTD = DIM // 8
UNITS_PER_H = NTB // TB_PER_UNIT
OUT_ROWS = B * DIM // 128


def _make_emb():
    mesh = plsc.VectorSubcoreMesh(core_axis_name="c", subcore_axis_name="s")

    @functools.partial(
        pl.kernel,
        mesh=mesh,
        out_type=jax.ShapeDtypeStruct((OUT_ROWS, 128), jnp.float32),
        scratch_types=[
            pltpu.VMEM((B_PER_W,), jnp.int32),
            pltpu.VMEM((2, UNIT, DIM), jnp.float32),
            pltpu.VMEM((2, NTD, 8, 129), jnp.float32),
            pltpu.SemaphoreType.DMA,
            pltpu.SemaphoreType.DMA,
            pltpu.SemaphoreType.DMA,
            pltpu.SemaphoreType.DMA,
            pltpu.SemaphoreType.DMA,
        ],
        compiler_params=pltpu.CompilerParams(
            use_tc_tiling_on_sc=False, needs_layout_passes=False
        ),
    )
    def emb(idx_hbm, table_hbm, out_hbm, idx_all, grows, trows, isem, g0, g1, o0, o1):
        gsems = (g0, g1)
        osems = (o0, o1)
        wid = lax.axis_index("s") * NC + lax.axis_index("c")
        wbase = wid * B_PER_W
        ubase = wid * NUNITS

        pltpu.async_copy(idx_hbm.at[pl.ds(wbase, B_PER_W)], idx_all, isem).wait()

        def start_gather(j, buf):
            pltpu.async_copy(
                table_hbm.at[idx_all.at[pl.ds(j * UNIT, UNIT)]],
                grows.at[buf],
                gsems[buf],
            )

        def wait_gather(buf):
            pltpu.make_async_copy(
                table_hbm.at[idx_all.at[pl.ds(0, UNIT)]],
                grows.at[buf],
                gsems[buf],
            ).wait()

        def transpose(buf):
            t_ref = trows.at[buf]
            iota = lax.iota(jnp.int32, 16)
            td_lo = jnp.bitwise_and(jnp.right_shift(iota, 3), 1)
            td_hi = td_lo + 2
            dmod = jnp.bitwise_and(iota, 7)
            zv = jnp.bitwise_and(iota, 0)
            for tbl in range(TB_PER_UNIT):
                rowv = dmod + (tbl * 8)
                for c in range(128):
                    t = tbl * 128 + c
                    cv = zv + c
                    v0 = grows[buf, t, pl.ds(0, 16)]
                    plsc.store_scatter(t_ref, [td_lo, rowv, cv], v0)
                    v1 = grows[buf, t, pl.ds(16, 16)]
                    plsc.store_scatter(t_ref, [td_hi, rowv, cv], v1)

        def start_stores(j, buf):
            u = ubase + j
            h = u // UNITS_PER_H
            tbq = u % UNITS_PER_H
            for td in range(NTD):
                row0 = h * 1024 + td * 256 + tbq * (TB_PER_UNIT * 8)
                pltpu.async_copy(
                    trows.at[buf, td, pl.ds(0, TB_PER_UNIT * 8), pl.ds(0, 128)],
                    out_hbm.at[pl.ds(row0, TB_PER_UNIT * 8)],
                    osems[buf],
                )

        def wait_stores(buf):
            for _ in range(NTD):
                pltpu.make_async_copy(
                    trows.at[buf, 0, pl.ds(0, TB_PER_UNIT * 8), pl.ds(0, 128)],
                    out_hbm.at[pl.ds(0, TB_PER_UNIT * 8)],
                    osems[buf],
                ).wait()

        start_gather(0, 0)
        start_gather(1, 1)

        def body(jj, carry):
            for buf in range(2):
                j = 2 * jj + buf
                wait_gather(buf)

                @pl.when(jj >= 1)
                def _():
                    wait_stores(buf)

                transpose(buf)

                @pl.when(jj < NUNITS // 2 - 1)
                def _():
                    start_gather(j + 2, buf)

                start_stores(j, buf)
            return carry

        lax.fori_loop(0, NUNITS // 2, body, 0)

        wait_stores(0)
        wait_stores(1)

    return emb


_emb = _make_emb()

def kernel(tokens, table):
    idx = tokens.T.reshape(B).astype(jnp.int32)
    out = _emb(idx, table)
    out6 = out.reshape(HIST, NTD, NTB, 8, 128)
    return out6.transpose(2, 4, 0, 1, 3).reshape(BATCH, HIST, DIM)
